# Initial kernel scaffold; baseline (speedup 1.0000x reference)
import jax
import jax.numpy as jnp
from jax.experimental import pallas as pl


def kernel(x, table):
    raise NotImplementedError("write your pallas kernel here")


# speedup vs baseline: 3.6264x; 3.6264x over previous
import jax
import jax.numpy as jnp
from jax.experimental import pallas as pl


def _add_kernel(x_ref, t_ref, o_ref):
    o_ref[...] = x_ref[...] + t_ref[...][None, :, :]


def kernel(x, table):
    B, L, D = x.shape
    TILE = 1024
    grid = (L // TILE,)
    return pl.pallas_call(
        _add_kernel,
        grid=grid,
        in_specs=[
            pl.BlockSpec((B, TILE, D), lambda i: (0, i, 0)),
            pl.BlockSpec((TILE, D), lambda i: (i, 0)),
        ],
        out_specs=pl.BlockSpec((B, TILE, D), lambda i: (0, i, 0)),
        out_shape=jax.ShapeDtypeStruct((B, L, D), x.dtype),
    )(x, table)
